# baseline (speedup 1.0000x reference)
import functools

import jax
import jax.numpy as jnp
from jax import lax
from jax.experimental import pallas as pl
from jax.experimental.pallas import tpu as pltpu
from jax.experimental.pallas import tpu_sc as plsc

_N = 8


def _gather_kernel(B, D, b_per_w, chunk, n_chunks, NC):
    mesh = plsc.VectorSubcoreMesh(core_axis_name="c", subcore_axis_name="s")
    N = _N
    n_sr = n_chunks // N

    @functools.partial(
        pl.kernel,
        mesh=mesh,
        out_type=jax.ShapeDtypeStruct((B, D), jnp.float32),
        compiler_params=pltpu.CompilerParams(use_tc_tiling_on_sc=False),
        scratch_types=[
            pltpu.VMEM((b_per_w,), jnp.int32),
            pltpu.VMEM((N, chunk, D), jnp.float32),
            pltpu.SemaphoreType.DMA,
            pltpu.SemaphoreType.DMA,
        ],
    )
    def k(tab_hbm, idx_hbm, out_hbm, idx_v, buf, gsem, ssem):
        wid = lax.axis_index("s") * NC + lax.axis_index("c")
        base = wid * b_per_w
        pltpu.sync_copy(idx_hbm.at[pl.ds(base, b_per_w)], idx_v)

        def gather(c, slot):
            return pltpu.make_async_copy(
                tab_hbm.at[idx_v.at[pl.ds(c * chunk, chunk)]],
                buf.at[slot],
                gsem,
            )

        def store(c, slot):
            return pltpu.make_async_copy(
                buf.at[slot],
                out_hbm.at[pl.ds(base + c * chunk, chunk)],
                ssem,
            )

        for j in range(N - 1):
            gather(j, j).start()

        gather(0, 0).wait()
        store(0, 0).start()
        gather(N - 1, N - 1).start()
        for j in range(1, N):
            gather(j, j).wait()
            store(j, j).start()
            store(j - 1, j - 1).wait()
            gather(j + N - 1, j - 1).start()

        def body(o, carry):
            c0 = o * N
            for j in range(N):
                c = c0 + j
                gather(c, j).wait()
                store(c, j).start()
                store(c - 1, (j - 1) % N).wait()
                gather(c + N - 1, (j - 1) % N).start()
            return carry

        lax.fori_loop(1, n_sr - 1, body, 0)

        c0 = (n_sr - 1) * N
        gather(c0, 0).wait()
        store(c0, 0).start()
        store(c0 - 1, N - 1).wait()
        gather(c0 + N - 1, N - 1).start()
        for j in range(1, N):
            c = c0 + j
            gather(c, j).wait()
            store(c, j).start()
        for j in range(N):
            store(c0 + j, j).wait()

    return k


def kernel(x, table):
    B0, S = x.shape
    V, D = table.shape
    B = B0 * S
    idx = x.reshape(B).astype(jnp.int32)

    info = plsc.get_sparse_core_info()
    NC, NS = info.num_cores, info.num_subcores
    NW = NC * NS
    b_per_w = B // NW
    chunk = 128
    n_chunks = b_per_w // chunk

    out = _gather_kernel(B, D, b_per_w, chunk, n_chunks, NC)(table, idx)
    return out.reshape(B0, S, D)


# speedup vs baseline: 1.0447x; 1.0447x over previous
import functools

import jax
import jax.numpy as jnp
from jax import lax
from jax.experimental import pallas as pl
from jax.experimental.pallas import tpu as pltpu
from jax.experimental.pallas import tpu_sc as plsc

_N = 8


def _gather_kernel(B, D, b_per_w, chunk, n_chunks, NC):
    mesh = plsc.VectorSubcoreMesh(core_axis_name="c", subcore_axis_name="s")
    N = _N
    n_sr = n_chunks // N

    @functools.partial(
        pl.kernel,
        mesh=mesh,
        out_type=jax.ShapeDtypeStruct((B, D), jnp.float32),
        compiler_params=pltpu.CompilerParams(use_tc_tiling_on_sc=False),
        scratch_types=[
            pltpu.VMEM((b_per_w,), jnp.int32),
            pltpu.VMEM((N, chunk, D), jnp.float32),
            pltpu.SemaphoreType.DMA,
            pltpu.SemaphoreType.DMA,
        ],
    )
    def k(tab_hbm, idx_hbm, out_hbm, idx_v, buf, gsem, ssem):
        wid = lax.axis_index("s") * NC + lax.axis_index("c")
        base = wid * b_per_w
        pltpu.sync_copy(idx_hbm.at[pl.ds(base, b_per_w)], idx_v)

        def gather(c, slot):
            return pltpu.make_async_copy(
                tab_hbm.at[idx_v.at[pl.ds(c * chunk, chunk)]],
                buf.at[slot],
                gsem,
            )

        def store(c, slot):
            return pltpu.make_async_copy(
                buf.at[slot],
                out_hbm.at[pl.ds(base + c * chunk, chunk)],
                ssem,
            )

        for j in range(N - 1):
            gather(j, j).start()

        gather(0, 0).wait()
        store(0, 0).start()
        gather(N - 1, N - 1).start()
        for j in range(1, N):
            gather(j, j).wait()
            store(j, j).start()
            store(j - 1, j - 1).wait()
            gather(j + N - 1, j - 1).start()

        def body(o, carry):
            c0 = o * N
            for j in range(N):
                c = c0 + j
                gather(c, j).wait()
                store(c, j).start()
                store(c - 1, (j - 1) % N).wait()
                gather(c + N - 1, (j - 1) % N).start()
            return carry

        lax.fori_loop(1, n_sr - 1, body, 0)

        c0 = (n_sr - 1) * N
        gather(c0, 0).wait()
        store(c0, 0).start()
        store(c0 - 1, N - 1).wait()
        gather(c0 + N - 1, N - 1).start()
        for j in range(1, N):
            c = c0 + j
            gather(c, j).wait()
            store(c, j).start()
        for j in range(N):
            store(c0 + j, j).wait()

    return k


def kernel(x, table):
    B0, S = x.shape
    V, D = table.shape
    B = B0 * S
    idx = jnp.swapaxes(x, 0, 1).reshape(B).astype(jnp.int32)

    info = plsc.get_sparse_core_info()
    NC, NS = info.num_cores, info.num_subcores
    NW = NC * NS
    b_per_w = B // NW
    chunk = 128
    n_chunks = b_per_w // chunk

    out = _gather_kernel(B, D, b_per_w, chunk, n_chunks, NC)(table, idx)
    return jnp.swapaxes(out.reshape(S, B0, D), 0, 1)


# speedup vs baseline: 1.2862x; 1.2312x over previous
import functools

import jax
import jax.numpy as jnp
from jax import lax
from jax.experimental import pallas as pl
from jax.experimental.pallas import tpu as pltpu
from jax.experimental.pallas import tpu_sc as plsc

_N = 5


def _gather_kernel(B, W, b_per_w, chunk, n_chunks, NC):
    mesh = plsc.VectorSubcoreMesh(core_axis_name="c", subcore_axis_name="s")
    N = _N
    n_sr = n_chunks // N

    @functools.partial(
        pl.kernel,
        mesh=mesh,
        out_type=jax.ShapeDtypeStruct((B, W), jnp.float32),
        compiler_params=pltpu.CompilerParams(use_tc_tiling_on_sc=False),
        scratch_types=[
            pltpu.VMEM((b_per_w,), jnp.int32),
            pltpu.VMEM((N, chunk, W), jnp.float32),
            pltpu.SemaphoreType.DMA,
            pltpu.SemaphoreType.DMA,
        ],
    )
    def k(tab_hbm, idx_hbm, out_hbm, idx_v, buf, gsem, ssem):
        wid = lax.axis_index("s") * NC + lax.axis_index("c")
        base = wid * b_per_w
        pltpu.sync_copy(idx_hbm.at[pl.ds(base, b_per_w)], idx_v)

        def gather(c, slot):
            return pltpu.make_async_copy(
                tab_hbm.at[idx_v.at[pl.ds(c * chunk, chunk)]],
                buf.at[slot],
                gsem,
            )

        def store(c, slot):
            return pltpu.make_async_copy(
                buf.at[slot],
                out_hbm.at[pl.ds(base + c * chunk, chunk)],
                ssem,
            )

        for j in range(N - 1):
            gather(j, j).start()

        gather(0, 0).wait()
        store(0, 0).start()
        gather(N - 1, N - 1).start()
        for j in range(1, N):
            gather(j, j).wait()
            store(j, j).start()
            store(j - 1, j - 1).wait()
            gather(j + N - 1, j - 1).start()

        def body(o, carry):
            c0 = o * N
            for j in range(N):
                c = c0 + j
                gather(c, j).wait()
                store(c, j).start()
                store(c - 1, (j - 1) % N).wait()
                gather(c + N - 1, (j - 1) % N).start()
            return carry

        lax.fori_loop(1, n_sr - 1, body, 0)

        c0 = (n_sr - 1) * N
        gather(c0, 0).wait()
        store(c0, 0).start()
        store(c0 - 1, N - 1).wait()
        gather(c0 + N - 1, N - 1).start()
        for j in range(1, N):
            c = c0 + j
            gather(c, j).wait()
            store(c, j).start()
        for j in range(N):
            store(c0 + j, j).wait()

    return k


def kernel(x, table):
    B0, S = x.shape
    V, D = table.shape
    B = B0 * S
    W = 128
    idx = jnp.swapaxes(x, 0, 1).reshape(B).astype(jnp.int32)
    tab_p = jnp.pad(table, ((0, 0), (0, W - D)))

    info = plsc.get_sparse_core_info()
    NC, NS = info.num_cores, info.num_subcores
    NW = NC * NS
    b_per_w = B // NW
    chunk = 128
    n_chunks = b_per_w // chunk

    out = _gather_kernel(B, W, b_per_w, chunk, n_chunks, NC)(tab_p, idx)
    return jnp.swapaxes(out.reshape(S, B0, W)[:, :, :D], 0, 1)


# speedup vs baseline: 1.3935x; 1.0834x over previous
import functools

import jax
import jax.numpy as jnp
from jax import lax
from jax.experimental import pallas as pl
from jax.experimental.pallas import tpu as pltpu
from jax.experimental.pallas import tpu_sc as plsc

_N = 5


def _gather_kernel(B, W, b_per_w, chunk, n_chunks, NC):
    mesh = plsc.VectorSubcoreMesh(core_axis_name="c", subcore_axis_name="s")
    N = _N
    n_sr = n_chunks // N

    @functools.partial(
        pl.kernel,
        mesh=mesh,
        out_type=jax.ShapeDtypeStruct((B, W), jnp.float32),
        compiler_params=pltpu.CompilerParams(use_tc_tiling_on_sc=False),
        scratch_types=[
            pltpu.VMEM((b_per_w,), jnp.int32),
            pltpu.VMEM((N, chunk, W), jnp.float32),
            pltpu.SemaphoreType.DMA,
            pltpu.SemaphoreType.DMA,
        ],
    )
    def k(tab_hbm, idx_hbm, out_hbm, idx_v, buf, gsem, ssem):
        wid = lax.axis_index("s") * NC + lax.axis_index("c")
        base = wid * b_per_w
        pltpu.sync_copy(idx_hbm.at[pl.ds(base, b_per_w)], idx_v)

        def gather(c, slot):
            return pltpu.make_async_copy(
                tab_hbm.at[idx_v.at[pl.ds(c * chunk, chunk)]],
                buf.at[slot],
                gsem,
            )

        def store(c, slot):
            return pltpu.make_async_copy(
                buf.at[slot, :, pl.ds(0, 64)],
                out_hbm.at[pl.ds(base + c * chunk, chunk), pl.ds(0, 64)],
                ssem,
            )

        for j in range(N - 1):
            gather(j, j).start()

        gather(0, 0).wait()
        store(0, 0).start()
        gather(N - 1, N - 1).start()
        for j in range(1, N):
            gather(j, j).wait()
            store(j, j).start()
            store(j - 1, j - 1).wait()
            gather(j + N - 1, j - 1).start()

        def body(o, carry):
            c0 = o * N
            for j in range(N):
                c = c0 + j
                gather(c, j).wait()
                store(c, j).start()
                store(c - 1, (j - 1) % N).wait()
                gather(c + N - 1, (j - 1) % N).start()
            return carry

        lax.fori_loop(1, n_sr - 1, body, 0)

        c0 = (n_sr - 1) * N
        gather(c0, 0).wait()
        store(c0, 0).start()
        store(c0 - 1, N - 1).wait()
        gather(c0 + N - 1, N - 1).start()
        for j in range(1, N):
            c = c0 + j
            gather(c, j).wait()
            store(c, j).start()
        for j in range(N):
            store(c0 + j, j).wait()

    return k


def kernel(x, table):
    B0, S = x.shape
    V, D = table.shape
    B = B0 * S
    W = 128
    idx = jnp.swapaxes(x, 0, 1).reshape(B).astype(jnp.int32)
    tab_p = jnp.pad(table, ((0, 0), (0, W - D)))

    info = plsc.get_sparse_core_info()
    NC, NS = info.num_cores, info.num_subcores
    NW = NC * NS
    b_per_w = B // NW
    chunk = 128
    n_chunks = b_per_w // chunk

    out = _gather_kernel(B, W, b_per_w, chunk, n_chunks, NC)(tab_p, idx)
    return jnp.swapaxes(out.reshape(S, B0, W)[:, :, :D], 0, 1)
